# baseline (speedup 1.0000x reference)
import functools

import jax
import jax.numpy as jnp
from jax import lax
from jax.experimental import pallas as pl
from jax.experimental.pallas import tpu as pltpu
from jax.experimental.pallas import tpu_sc as plsc

B, H, W, D = 256, 25, 25, 64
HW = H * W
HWP = 640
NPROP = 10
OFF = (0, 4, 12, 20, 24, 30, 36, 40, 48, 56)
SZ = (4, 8, 8, 4, 6, 6, 4, 8, 8, 4)
PAIRS = ((0, 1), (2, 3), (4, 5), (6, 7), (8, 9))
GROUP_PAIRS = ((0, 1), (2,), (3, 4))
NS_PAIR = (45, 45, 49, 45, 45)
BASES = (0, 2880, 5760, 8896, 11776)
TAB_LEN = 14656
ACC_LEN = D * HW
OUT_BATCH = 3 * ACC_LEN
NW = 32
ROUNDS = B // NW
NPV = HWP // 16


def _build_pair_tables(emb):
    zero = jnp.zeros((1, D), jnp.float32)
    parts = []
    for (pa, pb) in PAIRS:
        ra = jnp.concatenate([emb[OFF[pa]:OFF[pa] + SZ[pa]], zero])
        rb = jnp.concatenate([emb[OFF[pb]:OFF[pb] + SZ[pb]], zero])
        t = ra[:, None, :] + rb[None, :, :]
        n = (SZ[pa] + 1) * (SZ[pb] + 1)
        parts.append(t.reshape(n, D).T.reshape(-1))
    return jnp.concatenate(parts)


def _sc_body(tabs_hbm, idx_hbm, msk_hbm, out_hbm,
             tab_v, idx_v, msk_v, j_v, acc_a, acc_b, sem_a, sem_b):
    wid = lax.axis_index("s") * 2 + lax.axis_index("c")
    pltpu.sync_copy(tabs_hbm, tab_v)
    iota = lax.broadcasted_iota(jnp.int32, (16,), 0)
    tail_mask = iota < 1
    accs = (acc_a, acc_b, acc_a)
    sems = (sem_a, sem_b, sem_a)

    def round_body(r, carry):
        b = r * NW + wid
        pltpu.sync_copy(idx_hbm.at[b], idx_v)
        pltpu.sync_copy(msk_hbm.at[b], msk_v)

        @plsc.parallel_loop(0, NPV, step=1, unroll=2)
        def _(pv):
            base = pv * 16 + iota
            for q, (pa, pb) in enumerate(PAIRS):
                nA, nB = SZ[pa], SZ[pb]
                av = plsc.load_gather(idx_v, [pa * HWP + base])
                am = plsc.load_gather(msk_v, [pa * HWP + base])
                bv = plsc.load_gather(idx_v, [pb * HWP + base])
                bm = plsc.load_gather(msk_v, [pb * HWP + base])
                a_ = jnp.where(am > 0.5, av, nA)
                b_ = jnp.where(bm > 0.5, bv, nB)
                jv = a_ * (nB + 1) + b_ + BASES[q]
                plsc.store_scatter(j_v, [q * HWP + base], jv)

        for g in range(3):
            qs = GROUP_PAIRS[g]
            stride = NS_PAIR[qs[0]]
            acc_v, sem = accs[g], sems[g]
            dst = out_hbm.at[pl.ds(b * OUT_BATCH + g * ACC_LEN, ACC_LEN)]
            if g == 2:
                pltpu.make_async_copy(acc_v, dst, sem).wait()
            else:
                @pl.when(r >= 1)
                def _(acc_v=acc_v, dst=dst, sem=sem):
                    pltpu.make_async_copy(acc_v, dst, sem).wait()

            for chunk in range(5):
                jj = []
                for u in range(8):
                    pv = chunk * 8 + u
                    jj.append([plsc.load_gather(j_v, [q * HWP + pv * 16 + iota])
                               for q in qs])

                @plsc.parallel_loop(0, D, step=1, unroll=2)
                def _(c, jj=jj, qs=qs, chunk=chunk, stride=stride, acc_v=acc_v):
                    cn = c * stride
                    c625 = c * HW
                    for u in range(8):
                        pv = chunk * 8 + u
                        v = plsc.load_gather(tab_v, [cn + jj[u][0]])
                        if len(qs) == 2:
                            v = v + plsc.load_gather(tab_v, [cn + jj[u][1]])
                        saddr = c625 + pv * 16 + iota
                        if pv == NPV - 1:
                            plsc.store_scatter(acc_v, [saddr], v, mask=tail_mask)
                        else:
                            plsc.store_scatter(acc_v, [saddr], v)

            pltpu.async_copy(acc_v, dst, sem)
        return carry

    lax.fori_loop(0, ROUNDS, round_body, 0)
    pltpu.make_async_copy(acc_a, out_hbm.at[pl.ds(0, ACC_LEN)], sem_a).wait()
    pltpu.make_async_copy(acc_b, out_hbm.at[pl.ds(0, ACC_LEN)], sem_b).wait()


@jax.jit
def kernel(card_counts, card_colors, card_shapes, card_selections,
           leader_rotation, follower_rotation,
           prev_visited_card_counts, prev_visited_card_colors,
           prev_visited_card_shapes, prev_visited_card_selections,
           nonempty_property_mask, emb_table):
    props = (card_counts, card_colors, card_shapes, card_selections,
             leader_rotation, follower_rotation,
             prev_visited_card_counts, prev_visited_card_colors,
             prev_visited_card_shapes, prev_visited_card_selections)
    idx = jnp.stack([p.reshape(B, HW) for p in props], axis=1)
    idxp = jnp.zeros((B, NPROP, HWP), jnp.int32)
    idxp = idxp.at[:, :, :HW].set(idx).reshape(B, NPROP * HWP)
    mskp = jnp.zeros((B, NPROP, HWP), jnp.float32)
    mskp = mskp.at[:, :, :HW].set(
        nonempty_property_mask.reshape(B, NPROP, HW)).reshape(B, NPROP * HWP)
    tabs = _build_pair_tables(emb_table)

    mesh = plsc.VectorSubcoreMesh(core_axis_name="c", subcore_axis_name="s")
    out = pl.kernel(
        _sc_body,
        mesh=mesh,
        compiler_params=pltpu.CompilerParams(needs_layout_passes=False),
        out_type=jax.ShapeDtypeStruct((B * OUT_BATCH,), jnp.float32),
        scratch_types=[
            pltpu.VMEM((TAB_LEN,), jnp.float32),
            pltpu.VMEM((NPROP * HWP,), jnp.int32),
            pltpu.VMEM((NPROP * HWP,), jnp.float32),
            pltpu.VMEM((5 * HWP,), jnp.int32),
            pltpu.VMEM((ACC_LEN,), jnp.float32),
            pltpu.VMEM((ACC_LEN,), jnp.float32),
            pltpu.SemaphoreType.DMA,
            pltpu.SemaphoreType.DMA,
        ],
    )(tabs, idxp, mskp)
    return out.reshape(B, 3 * D, H, W)


# speedup vs baseline: 3.7840x; 3.7840x over previous
import functools

import jax
import jax.numpy as jnp
from jax import lax
from jax.experimental import pallas as pl
from jax.experimental.pallas import tpu as pltpu
from jax.experimental.pallas import tpu_sc as plsc

B, H, W, D = 256, 25, 25, 64
HW = H * W
HWP = 640
NPROP = 10
OFF = (0, 4, 12, 20, 24, 30, 36, 40, 48, 56)
SZ = (4, 8, 8, 4, 6, 6, 4, 8, 8, 4)
PAIRS = ((0, 1), (2, 3), (4, 5), (6, 7), (8, 9))
GROUP_PAIRS = ((0, 1), (2,), (3, 4))
NS_PAIR = (45, 45, 49, 45, 45)
BASES = (0, 2880, 5760, 8896, 11776)
TAB_LEN = 14656
ACC_LEN = D * HW
OUT_BATCH = 3 * ACC_LEN
NW = 32
NPV = HWP // 16
B_SC = 32
NB = 8
GROUP_ROWS = ((0, 24), (24, 36), (36, 60))


def _build_pair_tables(emb):
    zero = jnp.zeros((1, D), jnp.float32)
    parts = []
    for (pa, pb) in PAIRS:
        ra = jnp.concatenate([emb[OFF[pa]:OFF[pa] + SZ[pa]], zero])
        rb = jnp.concatenate([emb[OFF[pb]:OFF[pb] + SZ[pb]], zero])
        t = ra[:, None, :] + rb[None, :, :]
        n = (SZ[pa] + 1) * (SZ[pb] + 1)
        parts.append(t.reshape(n, D).T.reshape(-1))
    return jnp.concatenate(parts)


def _sc_body(tabs_hbm, idx_hbm, msk_hbm, out_hbm,
             tab_v, idx_v, msk_v, j_v, acc_a, acc_b, sem_a, sem_b):
    wid = lax.axis_index("s") * 2 + lax.axis_index("c")
    pltpu.sync_copy(tabs_hbm, tab_v)
    iota = lax.broadcasted_iota(jnp.int32, (16,), 0)
    tail_mask = iota < 1
    accs = (acc_a, acc_b, acc_a)
    sems = (sem_a, sem_b, sem_a)

    b = wid
    pltpu.sync_copy(idx_hbm.at[b], idx_v)
    pltpu.sync_copy(msk_hbm.at[b], msk_v)

    @plsc.parallel_loop(0, NPV, step=1, unroll=2)
    def _(pv):
        base = pv * 16 + iota
        for q, (pa, pb) in enumerate(PAIRS):
            nA, nB = SZ[pa], SZ[pb]
            av = plsc.load_gather(idx_v, [pa * HWP + base])
            am = plsc.load_gather(msk_v, [pa * HWP + base])
            bv = plsc.load_gather(idx_v, [pb * HWP + base])
            bm = plsc.load_gather(msk_v, [pb * HWP + base])
            a_ = jnp.where(am > 0.5, av, nA)
            b_ = jnp.where(bm > 0.5, bv, nB)
            jv = a_ * (nB + 1) + b_ + BASES[q]
            plsc.store_scatter(j_v, [q * HWP + base], jv)

    for g in range(3):
        qs = GROUP_PAIRS[g]
        stride = NS_PAIR[qs[0]]
        acc_v, sem = accs[g], sems[g]
        dst = out_hbm.at[pl.ds(b * OUT_BATCH + g * ACC_LEN, ACC_LEN)]
        if g == 2:
            pltpu.make_async_copy(acc_v, dst, sem).wait()
        for chunk in range(5):
            jj = []
            for u in range(8):
                pv = chunk * 8 + u
                jj.append([plsc.load_gather(j_v, [q * HWP + pv * 16 + iota])
                           for q in qs])

            @plsc.parallel_loop(0, D, step=1, unroll=2)
            def _(c, jj=jj, qs=qs, chunk=chunk, stride=stride, acc_v=acc_v):
                cn = c * stride
                c625 = c * HW
                for u in range(8):
                    pv = chunk * 8 + u
                    v = plsc.load_gather(tab_v, [cn + jj[u][0]])
                    if len(qs) == 2:
                        v = v + plsc.load_gather(tab_v, [cn + jj[u][1]])
                    saddr = c625 + pv * 16 + iota
                    if pv == NPV - 1:
                        plsc.store_scatter(acc_v, [saddr], v, mask=tail_mask)
                    else:
                        plsc.store_scatter(acc_v, [saddr], v)

        pltpu.async_copy(acc_v, dst, sem)

    pltpu.make_async_copy(acc_a, out_hbm.at[pl.ds(0, ACC_LEN)], sem_a).wait()
    pltpu.make_async_copy(acc_b, out_hbm.at[pl.ds(0, ACC_LEN)], sem_b).wait()


def _tc_block(idx_ref, mask_ref, t3t_ref, out_ref):
    row = lax.broadcasted_iota(jnp.int32, (D, HW), 0)
    t3t = t3t_ref[...]
    for b in range(NB):
        idx = idx_ref[b]
        mask = mask_ref[b]
        acc = jnp.zeros((D, HW), jnp.float32)
        for p in range(10):
            acc = acc + jnp.where(row == idx[p][None, :], mask[p][None, :], 0.0)
        out_ref[b] = jnp.dot(t3t, acc, preferred_element_type=jnp.float32)


@jax.jit
def kernel(card_counts, card_colors, card_shapes, card_selections,
           leader_rotation, follower_rotation,
           prev_visited_card_counts, prev_visited_card_colors,
           prev_visited_card_shapes, prev_visited_card_selections,
           nonempty_property_mask, emb_table):
    props = (card_counts, card_colors, card_shapes, card_selections,
             leader_rotation, follower_rotation,
             prev_visited_card_counts, prev_visited_card_colors,
             prev_visited_card_shapes, prev_visited_card_selections)
    idx = jnp.stack([p.reshape(B, HW) for p in props], axis=1)
    mask = nonempty_property_mask.reshape(B, NPROP, HW)

    idxp = jnp.zeros((B_SC, NPROP, HWP), jnp.int32)
    idxp = idxp.at[:, :, :HW].set(idx[:B_SC]).reshape(B_SC, NPROP * HWP)
    mskp = jnp.zeros((B_SC, NPROP, HWP), jnp.float32)
    mskp = mskp.at[:, :, :HW].set(mask[:B_SC]).reshape(B_SC, NPROP * HWP)
    tabs = _build_pair_tables(emb_table)

    mesh = plsc.VectorSubcoreMesh(core_axis_name="c", subcore_axis_name="s")
    out_sc = pl.kernel(
        _sc_body,
        mesh=mesh,
        compiler_params=pltpu.CompilerParams(needs_layout_passes=False),
        out_type=jax.ShapeDtypeStruct((B_SC * OUT_BATCH,), jnp.float32),
        scratch_types=[
            pltpu.VMEM((TAB_LEN,), jnp.float32),
            pltpu.VMEM((NPROP * HWP,), jnp.int32),
            pltpu.VMEM((NPROP * HWP,), jnp.float32),
            pltpu.VMEM((5 * HWP,), jnp.int32),
            pltpu.VMEM((ACC_LEN,), jnp.float32),
            pltpu.VMEM((ACC_LEN,), jnp.float32),
            pltpu.SemaphoreType.DMA,
            pltpu.SemaphoreType.DMA,
        ],
    )(tabs, idxp, mskp)

    idx_tc = idx + jnp.asarray(OFF, jnp.int32)[None, :, None]
    t3t = jnp.zeros((3 * D, D), jnp.float32)
    for g, (lo, hi) in enumerate(GROUP_ROWS):
        t3t = t3t.at[g * D:(g + 1) * D, lo:hi].set(emb_table[lo:hi].T)

    out_tc = pl.pallas_call(
        _tc_block,
        grid=((B - B_SC) // NB,),
        in_specs=[
            pl.BlockSpec((NB, NPROP, HW), lambda b: (b + B_SC // NB, 0, 0)),
            pl.BlockSpec((NB, NPROP, HW), lambda b: (b + B_SC // NB, 0, 0)),
            pl.BlockSpec((3 * D, D), lambda b: (0, 0)),
        ],
        out_specs=pl.BlockSpec((NB, 3 * D, HW), lambda b: (b + B_SC // NB, 0, 0)),
        out_shape=jax.ShapeDtypeStruct((B, 3 * D, HW), jnp.float32),
    )(idx_tc, mask, t3t)

    out = lax.dynamic_update_slice(
        out_tc, out_sc.reshape(B_SC, 3 * D, HW), (0, 0, 0))
    return out.reshape(B, 3 * D, H, W)


# speedup vs baseline: 3.8855x; 1.0268x over previous
import functools

import jax
import jax.numpy as jnp
from jax import lax
from jax.experimental import pallas as pl
from jax.experimental.pallas import tpu as pltpu
from jax.experimental.pallas import tpu_sc as plsc

B, H, W, D = 256, 25, 25, 64
HW = H * W
HWP = 640
NPROP = 10
OFF = (0, 4, 12, 20, 24, 30, 36, 40, 48, 56)
SZ = (4, 8, 8, 4, 6, 6, 4, 8, 8, 4)
PAIRS = ((0, 1), (2, 3), (4, 5), (6, 7), (8, 9))
GROUP_PAIRS = ((0, 1), (2,), (3, 4))
NS_PAIR = (45, 45, 49, 45, 45)
BASES = (0, 2880, 5760, 8896, 11776)
TAB_LEN = 14656
ACC_LEN = D * HW
OUT_BATCH = 3 * ACC_LEN
NW = 32
NPV = HWP // 16
B_SC = 32
NB = 16
GROUP_ROWS = ((0, 24), (24, 36), (36, 60))


def _build_pair_tables(emb):
    zero = jnp.zeros((1, D), jnp.float32)
    parts = []
    for (pa, pb) in PAIRS:
        ra = jnp.concatenate([emb[OFF[pa]:OFF[pa] + SZ[pa]], zero])
        rb = jnp.concatenate([emb[OFF[pb]:OFF[pb] + SZ[pb]], zero])
        t = ra[:, None, :] + rb[None, :, :]
        n = (SZ[pa] + 1) * (SZ[pb] + 1)
        parts.append(t.reshape(n, D).T.reshape(-1))
    return jnp.concatenate(parts)


def _sc_body(tabs_hbm, idx_hbm, msk_hbm, out_hbm,
             tab_v, idx_v, msk_v, j_v, acc_a, acc_b, sem_a, sem_b):
    wid = lax.axis_index("s") * 2 + lax.axis_index("c")
    pltpu.sync_copy(tabs_hbm, tab_v)
    iota = lax.broadcasted_iota(jnp.int32, (16,), 0)
    tail_mask = iota < 1
    accs = (acc_a, acc_b, acc_a)
    sems = (sem_a, sem_b, sem_a)

    b = wid
    pltpu.sync_copy(idx_hbm.at[b], idx_v)
    pltpu.sync_copy(msk_hbm.at[b], msk_v)

    @plsc.parallel_loop(0, NPV, step=1, unroll=2)
    def _(pv):
        base = pv * 16 + iota
        for q, (pa, pb) in enumerate(PAIRS):
            nA, nB = SZ[pa], SZ[pb]
            av = plsc.load_gather(idx_v, [pa * HWP + base])
            am = plsc.load_gather(msk_v, [pa * HWP + base])
            bv = plsc.load_gather(idx_v, [pb * HWP + base])
            bm = plsc.load_gather(msk_v, [pb * HWP + base])
            a_ = jnp.where(am > 0.5, av, nA)
            b_ = jnp.where(bm > 0.5, bv, nB)
            jv = a_ * (nB + 1) + b_ + BASES[q]
            plsc.store_scatter(j_v, [q * HWP + base], jv)

    for g in range(3):
        qs = GROUP_PAIRS[g]
        stride = NS_PAIR[qs[0]]
        acc_v, sem = accs[g], sems[g]
        dst = out_hbm.at[pl.ds(b * OUT_BATCH + g * ACC_LEN, ACC_LEN)]
        if g == 2:
            pltpu.make_async_copy(acc_v, dst, sem).wait()
        for chunk in range(5):
            jj = []
            for u in range(8):
                pv = chunk * 8 + u
                jj.append([plsc.load_gather(j_v, [q * HWP + pv * 16 + iota])
                           for q in qs])

            @plsc.parallel_loop(0, D, step=1, unroll=2)
            def _(c, jj=jj, qs=qs, chunk=chunk, stride=stride, acc_v=acc_v):
                cn = c * stride
                c625 = c * HW
                for u in range(8):
                    pv = chunk * 8 + u
                    v = plsc.load_gather(tab_v, [cn + jj[u][0]])
                    if len(qs) == 2:
                        v = v + plsc.load_gather(tab_v, [cn + jj[u][1]])
                    saddr = c625 + pv * 16 + iota
                    if pv == NPV - 1:
                        plsc.store_scatter(acc_v, [saddr], v, mask=tail_mask)
                    else:
                        plsc.store_scatter(acc_v, [saddr], v)

        pltpu.async_copy(acc_v, dst, sem)

    pltpu.make_async_copy(acc_a, out_hbm.at[pl.ds(0, ACC_LEN)], sem_a).wait()
    pltpu.make_async_copy(acc_b, out_hbm.at[pl.ds(0, ACC_LEN)], sem_b).wait()


def _tc_block(idx_ref, mask_ref, t3t_ref, out_ref):
    row = lax.broadcasted_iota(jnp.int32, (D, HW), 0)
    t3t = t3t_ref[...]
    for b in range(NB):
        idx = idx_ref[b]
        mask = mask_ref[b]
        acc = jnp.zeros((D, HW), jnp.float32)
        for p in range(10):
            acc = acc + jnp.where(row == idx[p][None, :], mask[p][None, :], 0.0)
        out_ref[b] = jnp.dot(t3t, acc, preferred_element_type=jnp.float32)


@jax.jit
def kernel(card_counts, card_colors, card_shapes, card_selections,
           leader_rotation, follower_rotation,
           prev_visited_card_counts, prev_visited_card_colors,
           prev_visited_card_shapes, prev_visited_card_selections,
           nonempty_property_mask, emb_table):
    props = (card_counts, card_colors, card_shapes, card_selections,
             leader_rotation, follower_rotation,
             prev_visited_card_counts, prev_visited_card_colors,
             prev_visited_card_shapes, prev_visited_card_selections)
    idx = jnp.stack([p.reshape(B, HW) for p in props], axis=1)
    mask = nonempty_property_mask.reshape(B, NPROP, HW)

    idxp = jnp.zeros((B_SC, NPROP, HWP), jnp.int32)
    idxp = idxp.at[:, :, :HW].set(idx[:B_SC]).reshape(B_SC, NPROP * HWP)
    mskp = jnp.zeros((B_SC, NPROP, HWP), jnp.float32)
    mskp = mskp.at[:, :, :HW].set(mask[:B_SC]).reshape(B_SC, NPROP * HWP)
    tabs = _build_pair_tables(emb_table)

    mesh = plsc.VectorSubcoreMesh(core_axis_name="c", subcore_axis_name="s")
    out_sc = pl.kernel(
        _sc_body,
        mesh=mesh,
        compiler_params=pltpu.CompilerParams(needs_layout_passes=False),
        out_type=jax.ShapeDtypeStruct((B_SC * OUT_BATCH,), jnp.float32),
        scratch_types=[
            pltpu.VMEM((TAB_LEN,), jnp.float32),
            pltpu.VMEM((NPROP * HWP,), jnp.int32),
            pltpu.VMEM((NPROP * HWP,), jnp.float32),
            pltpu.VMEM((5 * HWP,), jnp.int32),
            pltpu.VMEM((ACC_LEN,), jnp.float32),
            pltpu.VMEM((ACC_LEN,), jnp.float32),
            pltpu.SemaphoreType.DMA,
            pltpu.SemaphoreType.DMA,
        ],
    )(tabs, idxp, mskp)

    idx_tc = idx + jnp.asarray(OFF, jnp.int32)[None, :, None]
    t3t = jnp.zeros((3 * D, D), jnp.float32)
    for g, (lo, hi) in enumerate(GROUP_ROWS):
        t3t = t3t.at[g * D:(g + 1) * D, lo:hi].set(emb_table[lo:hi].T)

    out_tc = pl.pallas_call(
        _tc_block,
        grid=((B - B_SC) // NB,),
        in_specs=[
            pl.BlockSpec((NB, NPROP, HW), lambda b: (b + B_SC // NB, 0, 0)),
            pl.BlockSpec((NB, NPROP, HW), lambda b: (b + B_SC // NB, 0, 0)),
            pl.BlockSpec((3 * D, D), lambda b: (0, 0)),
        ],
        out_specs=pl.BlockSpec((NB, 3 * D, HW), lambda b: (b + B_SC // NB, 0, 0)),
        out_shape=jax.ShapeDtypeStruct((B, 3 * D, HW), jnp.float32),
    )(idx_tc, mask, t3t)

    out = lax.dynamic_update_slice(
        out_tc, out_sc.reshape(B_SC, 3 * D, HW), (0, 0, 0))
    return out.reshape(B, 3 * D, H, W)


# speedup vs baseline: 3.8895x; 1.0010x over previous
import functools

import jax
import jax.numpy as jnp
from jax import lax
from jax.experimental import pallas as pl
from jax.experimental.pallas import tpu as pltpu
from jax.experimental.pallas import tpu_sc as plsc

B, H, W, D = 256, 25, 25, 64
HW = H * W
HWP = 640
NPROP = 10
OFF = (0, 4, 12, 20, 24, 30, 36, 40, 48, 56)
SZ = (4, 8, 8, 4, 6, 6, 4, 8, 8, 4)
PAIRS = ((0, 1), (2, 3), (4, 5), (6, 7), (8, 9))
GROUP_PAIRS = ((0, 1), (2,), (3, 4))
NS_PAIR = (45, 45, 49, 45, 45)
BASES = (0, 2880, 5760, 8896, 11776)
TAB_LEN = 14656
ACC_LEN = D * HW
OUT_BATCH = 3 * ACC_LEN
NW = 32
NPV = HWP // 16
B_SC = 32
NB = 28
GROUP_ROWS = ((0, 24), (24, 36), (36, 60))


def _build_pair_tables(emb):
    zero = jnp.zeros((1, D), jnp.float32)
    parts = []
    for (pa, pb) in PAIRS:
        ra = jnp.concatenate([emb[OFF[pa]:OFF[pa] + SZ[pa]], zero])
        rb = jnp.concatenate([emb[OFF[pb]:OFF[pb] + SZ[pb]], zero])
        t = ra[:, None, :] + rb[None, :, :]
        n = (SZ[pa] + 1) * (SZ[pb] + 1)
        parts.append(t.reshape(n, D).T.reshape(-1))
    return jnp.concatenate(parts)


def _sc_body(tabs_hbm, idx_hbm, msk_hbm, out_hbm,
             tab_v, idx_v, msk_v, j_v, acc_a, acc_b, sem_a, sem_b):
    wid = lax.axis_index("s") * 2 + lax.axis_index("c")
    pltpu.sync_copy(tabs_hbm, tab_v)
    iota = lax.broadcasted_iota(jnp.int32, (16,), 0)
    tail_mask = iota < 1
    accs = (acc_a, acc_b, acc_a)
    sems = (sem_a, sem_b, sem_a)

    b = wid
    pltpu.sync_copy(idx_hbm.at[b], idx_v)
    pltpu.sync_copy(msk_hbm.at[b], msk_v)

    @plsc.parallel_loop(0, NPV, step=1, unroll=2)
    def _(pv):
        base = pv * 16 + iota
        for q, (pa, pb) in enumerate(PAIRS):
            nA, nB = SZ[pa], SZ[pb]
            av = plsc.load_gather(idx_v, [pa * HWP + base])
            am = plsc.load_gather(msk_v, [pa * HWP + base])
            bv = plsc.load_gather(idx_v, [pb * HWP + base])
            bm = plsc.load_gather(msk_v, [pb * HWP + base])
            a_ = jnp.where(am > 0.5, av, nA)
            b_ = jnp.where(bm > 0.5, bv, nB)
            jv = a_ * (nB + 1) + b_ + BASES[q]
            plsc.store_scatter(j_v, [q * HWP + base], jv)

    for g in range(3):
        qs = GROUP_PAIRS[g]
        stride = NS_PAIR[qs[0]]
        acc_v, sem = accs[g], sems[g]
        dst = out_hbm.at[pl.ds(b * OUT_BATCH + g * ACC_LEN, ACC_LEN)]
        if g == 2:
            pltpu.make_async_copy(acc_v, dst, sem).wait()
        for chunk in range(5):
            jj = []
            for u in range(8):
                pv = chunk * 8 + u
                jj.append([plsc.load_gather(j_v, [q * HWP + pv * 16 + iota])
                           for q in qs])

            @plsc.parallel_loop(0, D, step=1, unroll=2)
            def _(c, jj=jj, qs=qs, chunk=chunk, stride=stride, acc_v=acc_v):
                cn = c * stride
                c625 = c * HW
                for u in range(8):
                    pv = chunk * 8 + u
                    v = plsc.load_gather(tab_v, [cn + jj[u][0]])
                    if len(qs) == 2:
                        v = v + plsc.load_gather(tab_v, [cn + jj[u][1]])
                    saddr = c625 + pv * 16 + iota
                    if pv == NPV - 1:
                        plsc.store_scatter(acc_v, [saddr], v, mask=tail_mask)
                    else:
                        plsc.store_scatter(acc_v, [saddr], v)

        pltpu.async_copy(acc_v, dst, sem)

    pltpu.make_async_copy(acc_a, out_hbm.at[pl.ds(0, ACC_LEN)], sem_a).wait()
    pltpu.make_async_copy(acc_b, out_hbm.at[pl.ds(0, ACC_LEN)], sem_b).wait()


def _tc_block(idx_ref, mask_ref, t3t_ref, out_ref):
    row = lax.broadcasted_iota(jnp.int32, (D, HW), 0)
    t3t = t3t_ref[...]
    for b in range(NB):
        idx = idx_ref[b]
        mask = mask_ref[b]
        acc = jnp.zeros((D, HW), jnp.float32)
        for p in range(10):
            acc = acc + jnp.where(row == idx[p][None, :], mask[p][None, :], 0.0)
        out_ref[b] = jnp.dot(t3t, acc, preferred_element_type=jnp.float32)


@jax.jit
def kernel(card_counts, card_colors, card_shapes, card_selections,
           leader_rotation, follower_rotation,
           prev_visited_card_counts, prev_visited_card_colors,
           prev_visited_card_shapes, prev_visited_card_selections,
           nonempty_property_mask, emb_table):
    props = (card_counts, card_colors, card_shapes, card_selections,
             leader_rotation, follower_rotation,
             prev_visited_card_counts, prev_visited_card_colors,
             prev_visited_card_shapes, prev_visited_card_selections)
    idx = jnp.stack([p.reshape(B, HW) for p in props], axis=1)
    mask = nonempty_property_mask.reshape(B, NPROP, HW)

    idxp = jnp.zeros((B_SC, NPROP, HWP), jnp.int32)
    idxp = idxp.at[:, :, :HW].set(idx[:B_SC]).reshape(B_SC, NPROP * HWP)
    mskp = jnp.zeros((B_SC, NPROP, HWP), jnp.float32)
    mskp = mskp.at[:, :, :HW].set(mask[:B_SC]).reshape(B_SC, NPROP * HWP)
    tabs = _build_pair_tables(emb_table)

    mesh = plsc.VectorSubcoreMesh(core_axis_name="c", subcore_axis_name="s")
    out_sc = pl.kernel(
        _sc_body,
        mesh=mesh,
        compiler_params=pltpu.CompilerParams(needs_layout_passes=False),
        out_type=jax.ShapeDtypeStruct((B_SC * OUT_BATCH,), jnp.float32),
        scratch_types=[
            pltpu.VMEM((TAB_LEN,), jnp.float32),
            pltpu.VMEM((NPROP * HWP,), jnp.int32),
            pltpu.VMEM((NPROP * HWP,), jnp.float32),
            pltpu.VMEM((5 * HWP,), jnp.int32),
            pltpu.VMEM((ACC_LEN,), jnp.float32),
            pltpu.VMEM((ACC_LEN,), jnp.float32),
            pltpu.SemaphoreType.DMA,
            pltpu.SemaphoreType.DMA,
        ],
    )(tabs, idxp, mskp)

    idx_tc = idx + jnp.asarray(OFF, jnp.int32)[None, :, None]
    t3t = jnp.zeros((3 * D, D), jnp.float32)
    for g, (lo, hi) in enumerate(GROUP_ROWS):
        t3t = t3t.at[g * D:(g + 1) * D, lo:hi].set(emb_table[lo:hi].T)

    out_tc = pl.pallas_call(
        _tc_block,
        grid=((B - B_SC) // NB,),
        in_specs=[
            pl.BlockSpec((NB, NPROP, HW), lambda b: (b + B_SC // NB, 0, 0)),
            pl.BlockSpec((NB, NPROP, HW), lambda b: (b + B_SC // NB, 0, 0)),
            pl.BlockSpec((3 * D, D), lambda b: (0, 0)),
        ],
        out_specs=pl.BlockSpec((NB, 3 * D, HW), lambda b: (b + B_SC // NB, 0, 0)),
        out_shape=jax.ShapeDtypeStruct((B, 3 * D, HW), jnp.float32),
    )(idx_tc, mask, t3t)

    out = lax.dynamic_update_slice(
        out_tc, out_sc.reshape(B_SC, 3 * D, HW), (0, 0, 0))
    return out.reshape(B, 3 * D, H, W)


# speedup vs baseline: 4.0770x; 1.0482x over previous
import functools

import jax
import jax.numpy as jnp
from jax import lax
from jax.experimental import pallas as pl
from jax.experimental.pallas import tpu as pltpu
from jax.experimental.pallas import tpu_sc as plsc

B, H, W, D = 256, 25, 25, 64
HW = H * W
HWP = 640
NPROP = 10
OFF = (0, 4, 12, 20, 24, 30, 36, 40, 48, 56)
SZ = (4, 8, 8, 4, 6, 6, 4, 8, 8, 4)
PAIRS = ((0, 1), (2, 3), (4, 5), (6, 7), (8, 9))
GROUP_PAIRS = ((0, 1), (2,), (3, 4))
NS_PAIR = (45, 45, 49, 45, 45)
BASES = (0, 2880, 5760, 8896, 11776)
TAB_LEN = 14656
ACC_LEN = D * HW
OUT_BATCH = 3 * ACC_LEN
NW = 32
NPV = HWP // 16
B_SC = 32
NB = 16
GROUP_ROWS = ((0, 24), (24, 36), (36, 60))


def _build_pair_tables(emb):
    zero = jnp.zeros((1, D), jnp.float32)
    parts = []
    for (pa, pb) in PAIRS:
        ra = jnp.concatenate([emb[OFF[pa]:OFF[pa] + SZ[pa]], zero])
        rb = jnp.concatenate([emb[OFF[pb]:OFF[pb] + SZ[pb]], zero])
        t = ra[:, None, :] + rb[None, :, :]
        n = (SZ[pa] + 1) * (SZ[pb] + 1)
        parts.append(t.reshape(n, D).T.reshape(-1))
    return jnp.concatenate(parts)


def _sc_body(tabs_hbm, idx_hbm, msk_hbm, out_hbm,
             tab_v, idx_v, msk_v, j_v, acc_a, acc_b, sem_a, sem_b):
    wid = lax.axis_index("s") * 2 + lax.axis_index("c")
    pltpu.sync_copy(tabs_hbm, tab_v)
    iota = lax.broadcasted_iota(jnp.int32, (16,), 0)
    tail_mask = iota < 1
    accs = (acc_a, acc_b, acc_a)
    sems = (sem_a, sem_b, sem_a)

    b = wid
    pltpu.sync_copy(idx_hbm.at[b], idx_v)
    pltpu.sync_copy(msk_hbm.at[b], msk_v)

    @plsc.parallel_loop(0, NPV, step=1, unroll=2)
    def _(pv):
        base = pv * 16 + iota
        for q, (pa, pb) in enumerate(PAIRS):
            nA, nB = SZ[pa], SZ[pb]
            av = plsc.load_gather(idx_v, [pa * HWP + base])
            am = plsc.load_gather(msk_v, [pa * HWP + base])
            bv = plsc.load_gather(idx_v, [pb * HWP + base])
            bm = plsc.load_gather(msk_v, [pb * HWP + base])
            a_ = jnp.where(am > 0.5, av, nA)
            b_ = jnp.where(bm > 0.5, bv, nB)
            jv = a_ * (nB + 1) + b_ + BASES[q]
            plsc.store_scatter(j_v, [q * HWP + base], jv)

    for g in range(3):
        qs = GROUP_PAIRS[g]
        stride = NS_PAIR[qs[0]]
        acc_v, sem = accs[g], sems[g]
        dst = out_hbm.at[pl.ds(b * OUT_BATCH + g * ACC_LEN, ACC_LEN)]
        if g == 2:
            pltpu.make_async_copy(acc_v, dst, sem).wait()
        for chunk in range(5):
            jj = []
            for u in range(8):
                pv = chunk * 8 + u
                jj.append([plsc.load_gather(j_v, [q * HWP + pv * 16 + iota])
                           for q in qs])

            @plsc.parallel_loop(0, D, step=1, unroll=2)
            def _(c, jj=jj, qs=qs, chunk=chunk, stride=stride, acc_v=acc_v):
                cn = c * stride
                c625 = c * HW
                for u in range(8):
                    pv = chunk * 8 + u
                    v = plsc.load_gather(tab_v, [cn + jj[u][0]])
                    if len(qs) == 2:
                        v = v + plsc.load_gather(tab_v, [cn + jj[u][1]])
                    saddr = c625 + pv * 16 + iota
                    if pv == NPV - 1:
                        plsc.store_scatter(acc_v, [saddr], v, mask=tail_mask)
                    else:
                        plsc.store_scatter(acc_v, [saddr], v)

        pltpu.async_copy(acc_v, dst, sem)

    pltpu.make_async_copy(acc_a, out_hbm.at[pl.ds(0, ACC_LEN)], sem_a).wait()
    pltpu.make_async_copy(acc_b, out_hbm.at[pl.ds(0, ACC_LEN)], sem_b).wait()


def _tc_block(*refs):
    idx_refs = refs[:NPROP]
    mask_ref, t3t_ref, out_ref = refs[NPROP:]
    t3t = t3t_ref[...]
    rows = [lax.broadcasted_iota(jnp.int32, (D, HW), 0) - OFF[p]
            for p in range(NPROP)]
    for b in range(NB):
        mask = mask_ref[b]
        acc = jnp.zeros((D, HW), jnp.float32)
        for p in range(NPROP):
            acc = acc + jnp.where(rows[p] == idx_refs[p][b][None, :],
                                  mask[p][None, :], 0.0)
        out_ref[b] = jnp.dot(t3t, acc, preferred_element_type=jnp.float32)


@jax.jit
def kernel(card_counts, card_colors, card_shapes, card_selections,
           leader_rotation, follower_rotation,
           prev_visited_card_counts, prev_visited_card_colors,
           prev_visited_card_shapes, prev_visited_card_selections,
           nonempty_property_mask, emb_table):
    props = (card_counts, card_colors, card_shapes, card_selections,
             leader_rotation, follower_rotation,
             prev_visited_card_counts, prev_visited_card_colors,
             prev_visited_card_shapes, prev_visited_card_selections)
    props2d = [x.reshape(B, HW) for x in props]
    mask = nonempty_property_mask.reshape(B, NPROP, HW)

    idx_sc = jnp.stack([x[:B_SC] for x in props2d], axis=1)
    idxp = jnp.zeros((B_SC, NPROP, HWP), jnp.int32)
    idxp = idxp.at[:, :, :HW].set(idx_sc).reshape(B_SC, NPROP * HWP)
    mskp = jnp.zeros((B_SC, NPROP, HWP), jnp.float32)
    mskp = mskp.at[:, :, :HW].set(mask[:B_SC]).reshape(B_SC, NPROP * HWP)
    tabs = _build_pair_tables(emb_table)

    mesh = plsc.VectorSubcoreMesh(core_axis_name="c", subcore_axis_name="s")
    out_sc = pl.kernel(
        _sc_body,
        mesh=mesh,
        compiler_params=pltpu.CompilerParams(needs_layout_passes=False),
        out_type=jax.ShapeDtypeStruct((B_SC * OUT_BATCH,), jnp.float32),
        scratch_types=[
            pltpu.VMEM((TAB_LEN,), jnp.float32),
            pltpu.VMEM((NPROP * HWP,), jnp.int32),
            pltpu.VMEM((NPROP * HWP,), jnp.float32),
            pltpu.VMEM((5 * HWP,), jnp.int32),
            pltpu.VMEM((ACC_LEN,), jnp.float32),
            pltpu.VMEM((ACC_LEN,), jnp.float32),
            pltpu.SemaphoreType.DMA,
            pltpu.SemaphoreType.DMA,
        ],
    )(tabs, idxp, mskp)

    t3t = jnp.zeros((3 * D, D), jnp.float32)
    for g, (lo, hi) in enumerate(GROUP_ROWS):
        t3t = t3t.at[g * D:(g + 1) * D, lo:hi].set(emb_table[lo:hi].T)

    out_tc = pl.pallas_call(
        _tc_block,
        grid=((B - B_SC) // NB,),
        in_specs=(
            [pl.BlockSpec((NB, HW), lambda b: (b + B_SC // NB, 0))] * NPROP
            + [pl.BlockSpec((NB, NPROP, HW), lambda b: (b + B_SC // NB, 0, 0)),
               pl.BlockSpec((3 * D, D), lambda b: (0, 0))]
        ),
        out_specs=pl.BlockSpec((NB, 3 * D, HW), lambda b: (b + B_SC // NB, 0, 0)),
        out_shape=jax.ShapeDtypeStruct((B, 3 * D, HW), jnp.float32),
    )(*props2d, mask, t3t)

    out = lax.dynamic_update_slice(
        out_tc, out_sc.reshape(B_SC, 3 * D, HW), (0, 0, 0))
    return out.reshape(B, 3 * D, H, W)
